# baseline (speedup 1.0000x reference)
import functools

import jax
import jax.numpy as jnp
from jax import lax
from jax.experimental import pallas as pl
from jax.experimental.pallas import tpu as pltpu
from jax.experimental.pallas import tpu_sc as plsc

_VOCAB = 1000000
_D = 32
_BATCH = 16384
_FIELDS = 100
_NW = 32
_BW = _BATCH // _NW
_NG = 4
_NT = 2


def _emb_body(idx_hbm, table_hbm, out_hbm,
              ib0, ib1, ib2, ib3, r0, r1, r2, r3, t0, t1,
              si0, si1, si2, si3, sg0, sg1, sg2, sg3, sw0, sw1):
    wid = lax.axis_index("s") * 2 + lax.axis_index("c")
    b0 = wid * _BW
    ib = (ib0, ib1, ib2, ib3)
    rows = (r0, r1, r2, r3)
    tbuf = (t0, t1)
    si = (si0, si1, si2, si3)
    sg = (sg0, sg1, sg2, sg3)
    sw = (sw0, sw1)

    iota_lo = lax.iota(jnp.int32, 16)
    iota_hi = iota_lo + 16

    def idx_load(f, k):
        pltpu.async_copy(idx_hbm.at[f, pl.ds(b0, _BW)], ib[k], si[k])

    def wait_idx(k):
        pltpu.make_async_copy(idx_hbm.at[0, pl.ds(b0, _BW)], ib[k], si[k]).wait()

    def gather(k):
        pltpu.async_copy(table_hbm.at[ib[k]], rows[k], sg[k])

    def wait_gather(k):
        pltpu.make_async_copy(table_hbm.at[ib[0]], rows[k], sg[k]).wait()

    def writeback(f, b):
        pltpu.async_copy(tbuf[b], out_hbm.at[f, :, pl.ds(b0, _BW)], sw[b])

    def wait_writeback(b):
        pltpu.make_async_copy(
            tbuf[b], out_hbm.at[0, :, pl.ds(b0, _BW)], sw[b]).wait()

    def transpose(k, b):
        rv = rows[k]
        tv = tbuf[b]

        def tbody(j, carry):
            v0 = rv[j, pl.ds(0, 16)]
            v1 = rv[j, pl.ds(16, 16)]
            jj = jnp.full((16,), 0, jnp.int32) + j
            plsc.store_scatter(tv, [iota_lo, jj], v0)
            plsc.store_scatter(tv, [iota_hi, jj], v1)
            return carry

        lax.fori_loop(0, _BW, tbody, 0, unroll=4)

    for k in range(_NG):
        idx_load(k, k)
    for k in range(_NG):
        wait_idx(k)
        gather(k)

    for k in range(_NT):
        wait_gather(k)
        idx_load(k + _NG, k)
        transpose(k, k)
        wait_idx(k)
        gather(k)
        writeback(k, k)

    def wave(w, carry):
        base = _NT + 4 * w
        for k in range(_NG):
            f = base + k
            kk = (_NT + k) % _NG
            b = k % _NT
            wait_gather(kk)
            idx_load(f + _NG, kk)
            wait_writeback(b)
            transpose(kk, b)
            wait_idx(kk)
            gather(kk)
            writeback(f, b)
        return carry

    lax.fori_loop(0, 23, wave, 0)

    for f in range(94, 100):
        kk = f % _NG
        b = f % _NT
        wait_gather(kk)
        if f + _NG < _FIELDS:
            idx_load(f + _NG, kk)
        wait_writeback(b)
        transpose(kk, b)
        if f + _NG < _FIELDS:
            wait_idx(kk)
            gather(kk)
        writeback(f, b)
    for b in range(_NT):
        wait_writeback(b)


def kernel(input, weight):
    idx_fb = input.T
    mesh = plsc.VectorSubcoreMesh(core_axis_name="c", subcore_axis_name="s")
    out = pl.kernel(
        _emb_body,
        out_type=jax.ShapeDtypeStruct((_FIELDS, _D, _BATCH), jnp.float32),
        mesh=mesh,
        scratch_types=[
            pltpu.VMEM((_BW,), jnp.int32),
            pltpu.VMEM((_BW,), jnp.int32),
            pltpu.VMEM((_BW,), jnp.int32),
            pltpu.VMEM((_BW,), jnp.int32),
            pltpu.VMEM((_BW, _D), jnp.float32),
            pltpu.VMEM((_BW, _D), jnp.float32),
            pltpu.VMEM((_BW, _D), jnp.float32),
            pltpu.VMEM((_BW, _D), jnp.float32),
            pltpu.VMEM((_D, _BW), jnp.float32),
            pltpu.VMEM((_D, _BW), jnp.float32),
            pltpu.SemaphoreType.DMA,
            pltpu.SemaphoreType.DMA,
            pltpu.SemaphoreType.DMA,
            pltpu.SemaphoreType.DMA,
            pltpu.SemaphoreType.DMA,
            pltpu.SemaphoreType.DMA,
            pltpu.SemaphoreType.DMA,
            pltpu.SemaphoreType.DMA,
            pltpu.SemaphoreType.DMA,
            pltpu.SemaphoreType.DMA,
        ],
        compiler_params=pltpu.CompilerParams(
            use_tc_tiling_on_sc=False, needs_layout_passes=False),
    )(idx_fb, weight)
    return out.transpose(2, 0, 1)


# speedup vs baseline: 1.7573x; 1.7573x over previous
import functools

import jax
import jax.numpy as jnp
from jax import lax
from jax.experimental import pallas as pl
from jax.experimental.pallas import tpu as pltpu
from jax.experimental.pallas import tpu_sc as plsc

_VOCAB = 1000000
_D = 32
_BATCH = 16384
_FIELDS = 100
_NW = 32
_BW = _BATCH // _NW
_NG = 4
_NT = 2


def _emb_body(idx_hbm, table_hbm, out_hbm,
              ib0, ib1, ib2, ib3, r0, r1, r2, r3, t0, t1,
              si0, si1, si2, si3, sg0, sg1, sg2, sg3, sw0, sw1):
    wid = lax.axis_index("s") * 2 + lax.axis_index("c")
    b0 = wid * _BW
    ib = (ib0, ib1, ib2, ib3)
    rows = (r0, r1, r2, r3)
    tbuf = (t0, t1)
    si = (si0, si1, si2, si3)
    sg = (sg0, sg1, sg2, sg3)
    sw = (sw0, sw1)

    iota_lo = lax.iota(jnp.int32, 16)
    iota_hi = iota_lo + 16

    def idx_load(f, k):
        pltpu.async_copy(idx_hbm.at[f, pl.ds(b0, _BW)], ib[k], si[k])

    def wait_idx(k):
        pltpu.make_async_copy(idx_hbm.at[0, pl.ds(b0, _BW)], ib[k], si[k]).wait()

    def gather(k):
        pltpu.async_copy(table_hbm.at[ib[k]], rows[k], sg[k])

    def wait_gather(k):
        pltpu.make_async_copy(table_hbm.at[ib[0]], rows[k], sg[k]).wait()

    def writeback(f, b):
        pltpu.async_copy(
            tbuf[b].at[:, pl.ds(0, _BW)], out_hbm.at[f, :, pl.ds(b0, _BW)],
            sw[b])

    def wait_writeback(b):
        pltpu.make_async_copy(
            tbuf[b].at[:, pl.ds(0, _BW)], out_hbm.at[0, :, pl.ds(b0, _BW)],
            sw[b]).wait()

    def transpose(k, b):
        rv = rows[k]
        tv = tbuf[b]

        def tbody(j, jj):
            v0 = rv[j, pl.ds(0, 16)]
            v1 = rv[j, pl.ds(16, 16)]
            plsc.store_scatter(tv, [iota_lo, jj], v0)
            plsc.store_scatter(tv, [iota_hi, jj], v1)
            return jj + 1

        lax.fori_loop(0, _BW, tbody, iota_lo * 0, unroll=4)

    for k in range(_NG):
        idx_load(k, k)
    for k in range(_NG):
        wait_idx(k)
        gather(k)

    for k in range(_NT):
        wait_gather(k)
        idx_load(k + _NG, k)
        transpose(k, k)
        wait_idx(k)
        gather(k)
        writeback(k, k)

    def wave(w, carry):
        base = _NT + 4 * w
        for k in range(_NG):
            f = base + k
            kk = (_NT + k) % _NG
            b = k % _NT
            wait_gather(kk)
            idx_load(f + _NG, kk)
            wait_writeback(b)
            transpose(kk, b)
            wait_idx(kk)
            gather(kk)
            writeback(f, b)
        return carry

    lax.fori_loop(0, 23, wave, 0)

    for f in range(94, 100):
        kk = f % _NG
        b = f % _NT
        wait_gather(kk)
        if f + _NG < _FIELDS:
            idx_load(f + _NG, kk)
        wait_writeback(b)
        transpose(kk, b)
        if f + _NG < _FIELDS:
            wait_idx(kk)
            gather(kk)
        writeback(f, b)
    for b in range(_NT):
        wait_writeback(b)


def kernel(input, weight):
    idx_fb = input.T
    mesh = plsc.VectorSubcoreMesh(core_axis_name="c", subcore_axis_name="s")
    out = pl.kernel(
        _emb_body,
        out_type=jax.ShapeDtypeStruct((_FIELDS, _D, _BATCH), jnp.float32),
        mesh=mesh,
        scratch_types=[
            pltpu.VMEM((_BW,), jnp.int32),
            pltpu.VMEM((_BW,), jnp.int32),
            pltpu.VMEM((_BW,), jnp.int32),
            pltpu.VMEM((_BW,), jnp.int32),
            pltpu.VMEM((_BW, _D), jnp.float32),
            pltpu.VMEM((_BW, _D), jnp.float32),
            pltpu.VMEM((_BW, _D), jnp.float32),
            pltpu.VMEM((_BW, _D), jnp.float32),
            pltpu.VMEM((_D, _BW + 1), jnp.float32),
            pltpu.VMEM((_D, _BW + 1), jnp.float32),
            pltpu.SemaphoreType.DMA,
            pltpu.SemaphoreType.DMA,
            pltpu.SemaphoreType.DMA,
            pltpu.SemaphoreType.DMA,
            pltpu.SemaphoreType.DMA,
            pltpu.SemaphoreType.DMA,
            pltpu.SemaphoreType.DMA,
            pltpu.SemaphoreType.DMA,
            pltpu.SemaphoreType.DMA,
            pltpu.SemaphoreType.DMA,
        ],
        compiler_params=pltpu.CompilerParams(
            use_tc_tiling_on_sc=False, needs_layout_passes=False),
    )(idx_fb, weight)
    return out.transpose(2, 0, 1)


# speedup vs baseline: 2.1260x; 1.2098x over previous
import functools

import jax
import jax.numpy as jnp
from jax import lax
from jax.experimental import pallas as pl
from jax.experimental.pallas import tpu as pltpu
from jax.experimental.pallas import tpu_sc as plsc

_VOCAB = 1000000
_D = 32
_BATCH = 16384
_FIELDS = 100
_NW = 32
_BW = _BATCH // _NW
_NG = 4
_NT = 2
_TR = _D // 8
_TC = _BATCH // 128
_TCW = _BW // 128
_CS = 129


def _emb_body(idx_hbm, table_hbm, out_hbm,
              ib0, ib1, ib2, ib3, r0, r1, r2, r3, t0, t1,
              si0, si1, si2, si3, sg0, sg1, sg2, sg3, sw0, sw1):
    wid = lax.axis_index("s") * 2 + lax.axis_index("c")
    b0 = wid * _BW
    tc0 = wid * _TCW
    ib = (ib0, ib1, ib2, ib3)
    rows = (r0, r1, r2, r3)
    tbuf = (t0, t1)
    si = (si0, si1, si2, si3)
    sg = (sg0, sg1, sg2, sg3)
    sw = (sw0, sw1)

    iota = lax.iota(jnp.int32, 16)
    tr_lo = iota // 8
    r_lo = iota % 8
    tr_hi = (iota + 16) // 8
    r_hi = (iota + 16) % 8

    def idx_load(f, k):
        pltpu.async_copy(idx_hbm.at[f, pl.ds(b0, _BW)], ib[k], si[k])

    def wait_idx(k):
        pltpu.make_async_copy(idx_hbm.at[0, pl.ds(b0, _BW)], ib[k], si[k]).wait()

    def gather(k):
        pltpu.async_copy(table_hbm.at[ib[k]], rows[k], sg[k])

    def wait_gather(k):
        pltpu.make_async_copy(table_hbm.at[ib[0]], rows[k], sg[k]).wait()

    def writeback(f, b):
        pltpu.async_copy(
            tbuf[b].at[:, :, :, pl.ds(0, 128)],
            out_hbm.at[f, :, pl.ds(tc0, _TCW), :, :], sw[b])

    def wait_writeback(b):
        pltpu.make_async_copy(
            tbuf[b].at[:, :, :, pl.ds(0, 128)],
            out_hbm.at[0, :, pl.ds(tc0, _TCW), :, :], sw[b]).wait()

    def transpose(k, b):
        rv = rows[k]
        tv = tbuf[b]

        for tcl in range(_TCW):
            tcl_v = jnp.full((16,), tcl, jnp.int32)

            def tbody(j2, cc):
                j = tcl * 128 + j2
                v0 = rv[j, pl.ds(0, 16)]
                v1 = rv[j, pl.ds(16, 16)]
                plsc.store_scatter(tv, [tr_lo, tcl_v, r_lo, cc], v0)
                plsc.store_scatter(tv, [tr_hi, tcl_v, r_hi, cc], v1)
                return cc + 1

            lax.fori_loop(0, 128, tbody, iota * 0, unroll=4)

    for k in range(_NG):
        idx_load(k, k)
    for k in range(_NG):
        wait_idx(k)
        gather(k)

    for k in range(_NT):
        wait_gather(k)
        idx_load(k + _NG, k)
        transpose(k, k)
        wait_idx(k)
        gather(k)
        writeback(k, k)

    def wave(w, carry):
        base = _NT + 4 * w
        for k in range(_NG):
            f = base + k
            kk = (_NT + k) % _NG
            b = k % _NT
            wait_gather(kk)
            idx_load(f + _NG, kk)
            wait_writeback(b)
            transpose(kk, b)
            wait_idx(kk)
            gather(kk)
            writeback(f, b)
        return carry

    lax.fori_loop(0, 23, wave, 0)

    for f in range(94, 100):
        kk = f % _NG
        b = f % _NT
        wait_gather(kk)
        if f + _NG < _FIELDS:
            idx_load(f + _NG, kk)
        wait_writeback(b)
        transpose(kk, b)
        if f + _NG < _FIELDS:
            wait_idx(kk)
            gather(kk)
        writeback(f, b)
    for b in range(_NT):
        wait_writeback(b)


def kernel(input, weight):
    idx_fb = input.T
    mesh = plsc.VectorSubcoreMesh(core_axis_name="c", subcore_axis_name="s")
    out5 = pl.kernel(
        _emb_body,
        out_type=jax.ShapeDtypeStruct((_FIELDS, _TR, _TC, 8, 128),
                                      jnp.float32),
        mesh=mesh,
        scratch_types=[
            pltpu.VMEM((_BW,), jnp.int32),
            pltpu.VMEM((_BW,), jnp.int32),
            pltpu.VMEM((_BW,), jnp.int32),
            pltpu.VMEM((_BW,), jnp.int32),
            pltpu.VMEM((_BW, _D), jnp.float32),
            pltpu.VMEM((_BW, _D), jnp.float32),
            pltpu.VMEM((_BW, _D), jnp.float32),
            pltpu.VMEM((_BW, _D), jnp.float32),
            pltpu.VMEM((_TR, _TCW, 8, _CS), jnp.float32),
            pltpu.VMEM((_TR, _TCW, 8, _CS), jnp.float32),
            pltpu.SemaphoreType.DMA,
            pltpu.SemaphoreType.DMA,
            pltpu.SemaphoreType.DMA,
            pltpu.SemaphoreType.DMA,
            pltpu.SemaphoreType.DMA,
            pltpu.SemaphoreType.DMA,
            pltpu.SemaphoreType.DMA,
            pltpu.SemaphoreType.DMA,
            pltpu.SemaphoreType.DMA,
            pltpu.SemaphoreType.DMA,
        ],
        compiler_params=pltpu.CompilerParams(
            use_tc_tiling_on_sc=False, needs_layout_passes=False),
    )(idx_fb, weight)
    out = out5.transpose(0, 1, 3, 2, 4).reshape(_FIELDS, _D, _BATCH)
    return out.transpose(2, 0, 1)


# speedup vs baseline: 2.1351x; 1.0043x over previous
import functools

import jax
import jax.numpy as jnp
from jax import lax
from jax.experimental import pallas as pl
from jax.experimental.pallas import tpu as pltpu
from jax.experimental.pallas import tpu_sc as plsc

_VOCAB = 1000000
_D = 32
_BATCH = 16384
_FIELDS = 100
_NW = 32
_BW = _BATCH // _NW
_NG = 4
_NT = 2
_TR = _D // 8
_TC = _BATCH // 128
_TCW = _BW // 128
_CS = 129


def _emb_body(idx_hbm, table_hbm, out_hbm,
              ib0, ib1, ib2, ib3, r0, r1, r2, r3, t0, t1,
              si0, si1, si2, si3, sg0, sg1, sg2, sg3, sw0, sw1):
    wid = lax.axis_index("s") * 2 + lax.axis_index("c")
    b0 = wid * _BW
    tc0 = wid * _TCW
    ib = (ib0, ib1, ib2, ib3)
    rows = (r0, r1, r2, r3)
    tbuf = (t0, t1)
    si = (si0, si1, si2, si3)
    sg = (sg0, sg1, sg2, sg3)
    sw = (sw0, sw1)

    iota = lax.iota(jnp.int32, 16)
    tr_lo = iota // 8
    r_lo = iota % 8
    tr_hi = (iota + 16) // 8
    r_hi = (iota + 16) % 8

    def idx_load(f, k):
        pltpu.async_copy(idx_hbm.at[f, pl.ds(b0, _BW)], ib[k], si[k])

    def wait_idx(k):
        pltpu.make_async_copy(idx_hbm.at[0, pl.ds(b0, _BW)], ib[k], si[k]).wait()

    def gather(k):
        pltpu.async_copy(table_hbm.at[ib[k]], rows[k], sg[k])

    def wait_gather(k):
        pltpu.make_async_copy(table_hbm.at[ib[0]], rows[k], sg[k]).wait()

    def writeback(f, b):
        pltpu.async_copy(
            tbuf[b].at[:, :, :, pl.ds(0, 128)],
            out_hbm.at[f, :, pl.ds(tc0, _TCW), :, :], sw[b])

    def wait_writeback(b):
        pltpu.make_async_copy(
            tbuf[b].at[:, :, :, pl.ds(0, 128)],
            out_hbm.at[0, :, pl.ds(tc0, _TCW), :, :], sw[b]).wait()

    def transpose(k, b):
        rv = rows[k]
        tv = tbuf[b]

        for tcl in range(_TCW):
            tcl_v = jnp.full((16,), tcl, jnp.int32)

            def tbody(j2, cc):
                j = tcl * 128 + j2
                v0 = rv[j, pl.ds(0, 16)]
                v1 = rv[j, pl.ds(16, 16)]
                plsc.store_scatter(tv, [tr_lo, tcl_v, r_lo, cc], v0)
                plsc.store_scatter(tv, [tr_hi, tcl_v, r_hi, cc], v1)
                return cc + 1

            lax.fori_loop(0, 128, tbody, iota * 0, unroll=8)

    for k in range(_NG):
        idx_load(k, k)
    for k in range(_NG):
        wait_idx(k)
        gather(k)

    for k in range(_NT):
        wait_gather(k)
        idx_load(k + _NG, k)
        transpose(k, k)
        wait_idx(k)
        gather(k)
        writeback(k, k)

    def wave(w, carry):
        base = _NT + 4 * w
        for k in range(_NG):
            f = base + k
            kk = (_NT + k) % _NG
            b = k % _NT
            wait_gather(kk)
            idx_load(f + _NG, kk)
            wait_writeback(b)
            transpose(kk, b)
            wait_idx(kk)
            gather(kk)
            writeback(f, b)
        return carry

    lax.fori_loop(0, 23, wave, 0)

    for f in range(94, 100):
        kk = f % _NG
        b = f % _NT
        wait_gather(kk)
        if f + _NG < _FIELDS:
            idx_load(f + _NG, kk)
        wait_writeback(b)
        transpose(kk, b)
        if f + _NG < _FIELDS:
            wait_idx(kk)
            gather(kk)
        writeback(f, b)
    for b in range(_NT):
        wait_writeback(b)


def kernel(input, weight):
    idx_fb = input.T
    mesh = plsc.VectorSubcoreMesh(core_axis_name="c", subcore_axis_name="s")
    out5 = pl.kernel(
        _emb_body,
        out_type=jax.ShapeDtypeStruct((_FIELDS, _TR, _TC, 8, 128),
                                      jnp.float32),
        mesh=mesh,
        scratch_types=[
            pltpu.VMEM((_BW,), jnp.int32),
            pltpu.VMEM((_BW,), jnp.int32),
            pltpu.VMEM((_BW,), jnp.int32),
            pltpu.VMEM((_BW,), jnp.int32),
            pltpu.VMEM((_BW, _D), jnp.float32),
            pltpu.VMEM((_BW, _D), jnp.float32),
            pltpu.VMEM((_BW, _D), jnp.float32),
            pltpu.VMEM((_BW, _D), jnp.float32),
            pltpu.VMEM((_TR, _TCW, 8, _CS), jnp.float32),
            pltpu.VMEM((_TR, _TCW, 8, _CS), jnp.float32),
            pltpu.SemaphoreType.DMA,
            pltpu.SemaphoreType.DMA,
            pltpu.SemaphoreType.DMA,
            pltpu.SemaphoreType.DMA,
            pltpu.SemaphoreType.DMA,
            pltpu.SemaphoreType.DMA,
            pltpu.SemaphoreType.DMA,
            pltpu.SemaphoreType.DMA,
            pltpu.SemaphoreType.DMA,
            pltpu.SemaphoreType.DMA,
        ],
        compiler_params=pltpu.CompilerParams(
            use_tc_tiling_on_sc=False, needs_layout_passes=False),
    )(idx_fb, weight)
    out = out5.transpose(0, 1, 3, 2, 4).reshape(_FIELDS, _D, _BATCH)
    return out.transpose(2, 0, 1)
